# Initial kernel scaffold; baseline (speedup 1.0000x reference)
import jax
import jax.numpy as jnp
from jax.experimental import pallas as pl


def kernel(x, edge_index, W, b):
    raise NotImplementedError("write your pallas kernel here")


# speedup vs baseline: 14.6588x; 14.6588x over previous
import functools

import jax
import jax.numpy as jnp
from jax import lax
from jax.experimental import pallas as pl
from jax.experimental.pallas import tpu as pltpu
from jax.experimental.pallas import tpu_sc as plsc

N = 10000
F = 128
NP = 10112
E = 320000
RT = 80
ET = RT * 128
EP = 32 * ET
NC, NS = 2, 16
STRIPE = NP // NS
HB = 16384

_mesh = plsc.VectorSubcoreMesh(core_axis_name="c", subcore_axis_name="s")


@functools.partial(
    pl.kernel,
    out_type=jax.ShapeDtypeStruct((NC, HB), jnp.float32),
    mesh=_mesh,
    scratch_types=[
        pltpu.VMEM((RT, 128), jnp.int32),
        pltpu.VMEM((128,), jnp.float32),
        pltpu.VMEM_SHARED((HB,), jnp.float32),
    ],
)
def _deg_kernel(dst_hbm, zh_hbm, out_hbm, idx_v, ones_v, hist_sh):
    c = lax.axis_index("c")
    s = lax.axis_index("s")
    wid = c * NS + s
    pltpu.sync_copy(dst_hbm.at[pl.ds(wid * RT, RT)], idx_v)
    for i in range(8):
        ones_v[pl.ds(i * 16, 16)] = jnp.ones((16,), jnp.float32)
    pltpu.sync_copy(
        zh_hbm.at[pl.ds(s * (HB // NS), HB // NS)],
        hist_sh.at[pl.ds(s * (HB // NS), HB // NS)],
    )
    plsc.subcore_barrier()

    def body(j, carry):
        pltpu.sync_copy(ones_v, hist_sh.at[idx_v.at[j]], add=True)
        return carry

    lax.fori_loop(0, RT, body, 0)
    plsc.subcore_barrier()

    @pl.when(s == 0)
    def _():
        pltpu.sync_copy(hist_sh, out_hbm.at[c])


@functools.partial(
    pl.kernel,
    out_type=jax.ShapeDtypeStruct((NC, NP, F), jnp.float32),
    mesh=_mesh,
    scratch_types=[
        pltpu.VMEM((RT, 128), jnp.int32),
        pltpu.VMEM((RT, 128), jnp.int32),
        pltpu.VMEM((128, F), jnp.float32),
        pltpu.VMEM_SHARED((NP, F), jnp.float32),
        pltpu.SemaphoreType.DMA,
    ],
)
def _agg_kernel(y_hbm, src_hbm, dst_hbm, zeros_hbm, out_hbm, src_v, dst_v, rows_v, acc_sh, sem):
    c = lax.axis_index("c")
    s = lax.axis_index("s")
    wid = c * NS + s
    pltpu.sync_copy(src_hbm.at[pl.ds(wid * RT, RT)], src_v)
    pltpu.sync_copy(dst_hbm.at[pl.ds(wid * RT, RT)], dst_v)
    pltpu.sync_copy(
        zeros_hbm.at[pl.ds(s * STRIPE, STRIPE)], acc_sh.at[pl.ds(s * STRIPE, STRIPE)]
    )
    plsc.subcore_barrier()

    def body(j, carry):
        pltpu.async_copy(y_hbm.at[src_v.at[j]], rows_v, sem).wait()
        pltpu.sync_copy(rows_v, acc_sh.at[dst_v.at[j]], add=True)
        return carry

    lax.fori_loop(0, RT, body, 0)
    plsc.subcore_barrier()
    pltpu.sync_copy(
        acc_sh.at[pl.ds(s * STRIPE, STRIPE)], out_hbm.at[c, pl.ds(s * STRIPE, STRIPE)]
    )


def _mm_body(x_ref, w_ref, h_ref, y_ref, d_ref):
    deg = h_ref[0] + h_ref[1] + 1.0
    d = lax.rsqrt(deg)
    xw = jnp.dot(x_ref[...], w_ref[...], preferred_element_type=jnp.float32)
    y_ref[...] = xw * d[:NP]
    d_ref[...] = d


_mm = pl.pallas_call(
    _mm_body,
    out_shape=[
        jax.ShapeDtypeStruct((NP, F), jnp.float32),
        jax.ShapeDtypeStruct((HB, 1), jnp.float32),
    ],
)


def _fin_body(acc_ref, y_ref, d_ref, b_ref, relu_ref, out_ref):
    tot = acc_ref[0, :N, :] + acc_ref[1, :N, :] + y_ref[:N, :]
    o = d_ref[:N] * tot + b_ref[...]
    out_ref[...] = o
    relu_ref[...] = jnp.maximum(o, 0.0)


_fin = pl.pallas_call(
    _fin_body,
    out_shape=[
        jax.ShapeDtypeStruct((N, F), jnp.float32),
        jax.ShapeDtypeStruct((N, F), jnp.float32),
    ],
)


def kernel(x, edge_index, W, b):
    fill = jnp.full((EP - E,), N, jnp.int32)
    src_p = jnp.concatenate([edge_index[0], fill]).reshape(EP // 128, 128)
    dst_p = jnp.concatenate([edge_index[1], fill]).reshape(EP // 128, 128)
    x_p = jnp.pad(x, ((0, NP - N), (0, 0)))
    zeros_np = jnp.zeros((NP, F), jnp.float32)
    zh = jnp.zeros((HB,), jnp.float32)

    hist = _deg_kernel(dst_p, zh)
    h_r = hist.reshape(NC, HB, 1)
    y, dcol = _mm(x_p, W, h_r)
    acc = _agg_kernel(y, src_p, dst_p, zeros_np)
    relu_o, o = _fin(acc, y, dcol, b.reshape(1, F))
    return (relu_o, o)
